# Initial kernel scaffold; baseline (speedup 1.0000x reference)
import jax
import jax.numpy as jnp
from jax.experimental import pallas as pl


def kernel(solute_x, solute_edge_index, solute_e, solvent_x, solvent_edge_index, solvent_e, solute_len_matrix, solvent_len_matrix, params):
    raise NotImplementedError("write your pallas kernel here")


# speedup vs baseline: 12.8744x; 12.8744x over previous
import functools

import jax
import jax.numpy as jnp
from jax import lax
from jax.experimental import pallas as pl
from jax.experimental.pallas import tpu as pltpu
from jax.experimental.pallas import tpu_sc as plsc

N = 4096
E = 65536
DH = 4
NSTEP = 6
NS = 16
EPT = E // NS
NPT = N // NS
CW = 1024
NCH = EPT // CW
BE = 2048
BM = 512
BN = 512


def _edge_net_body(eT_ref, w1_ref, b1_ref, w2_ref, b2_ref, out_ref):
    h = jnp.maximum(
        jnp.dot(w1_ref[0], eT_ref[0], preferred_element_type=jnp.float32)
        + b1_ref[0], 0.0)
    out_ref[0] = (
        jnp.dot(w2_ref[0], h, preferred_element_type=jnp.float32) + b2_ref[0])


def _edge_net(eT, w1, b1, w2, b2):
    grid = (2, E // BE)
    return pl.pallas_call(
        _edge_net_body,
        grid=grid,
        in_specs=[
            pl.BlockSpec((1, 16, BE), lambda g, j: (g, 0, j)),
            pl.BlockSpec((1, 1024, 16), lambda g, j: (g, 0, 0)),
            pl.BlockSpec((1, 1024, 1), lambda g, j: (g, 0, 0)),
            pl.BlockSpec((1, 16, 1024), lambda g, j: (g, 0, 0)),
            pl.BlockSpec((1, 16, 1), lambda g, j: (g, 0, 0)),
        ],
        out_specs=pl.BlockSpec((1, 16, BE), lambda g, j: (g, 0, j)),
        out_shape=jax.ShapeDtypeStruct((2, 16, E), jnp.float32),
    )(eT, w1, b1, w2, b2)


def _lin0_body(xT_ref, w_ref, b_ref, out_ref):
    out_ref[0] = jnp.maximum(
        jnp.dot(w_ref[0], xT_ref[0], preferred_element_type=jnp.float32)
        + b_ref[0], 0.0)


def _lin0(xT, w, b):
    return pl.pallas_call(
        _lin0_body,
        grid=(2,),
        in_specs=[
            pl.BlockSpec((1, DH, N), lambda g: (g, 0, 0)),
            pl.BlockSpec((1, DH, DH), lambda g: (g, 0, 0)),
            pl.BlockSpec((1, DH, 1), lambda g: (g, 0, 0)),
        ],
        out_specs=pl.BlockSpec((1, DH, N), lambda g: (g, 0, 0)),
        out_shape=jax.ShapeDtypeStruct((2, DH, N), jnp.float32),
    )(xT, w, b)


def _splat(prm_v, i):
    return plsc.load_gather(prm_v, [jnp.full((16,), i, jnp.int32)])


def _mp_body(out0_hbm, xT_hbm, weT_hbm, eidx_hbm, prm_hbm, sf_hbm,
             src_v, dst_v, web0, web1, o0, o1, o2, o3, agg_v,
             red_v, acc_v, new_v, x_v, prm_v, sem0, sem1, agg_sh, out_sh):
    c = lax.axis_index("c")
    s = lax.axis_index("s")
    ebase = s * EPT
    nbase = s * NPT
    outs = [o0, o1, o2, o3]
    webs = [web0, web1]
    sems = [sem0, sem1]

    pltpu.sync_copy(eidx_hbm.at[c, 0, pl.ds(ebase, EPT)], src_v)
    pltpu.sync_copy(eidx_hbm.at[c, 1, pl.ds(ebase, EPT)], dst_v)
    for j in range(DH):
        pltpu.sync_copy(out0_hbm.at[c, j], outs[j])
    pltpu.sync_copy(xT_hbm.at[c, :, pl.ds(nbase, NPT)], x_v)
    pltpu.sync_copy(prm_hbm.at[c], prm_v)

    def fire(cb, b):
        pltpu.async_copy(
            weT_hbm.at[c, :, pl.ds(ebase + cb * CW, CW)], webs[b], sems[b])

    def wait(cb, b):
        pltpu.make_async_copy(
            weT_hbm.at[c, :, pl.ds(ebase + cb * CW, CW)], webs[b],
            sems[b]).wait()

    for step in range(NSTEP):
        last = step == NSTEP - 1

        def zbody(k, carry):
            z = jnp.zeros((16,), jnp.float32)
            for j in range(DH):
                agg_v[j, pl.ds(k * 16, 16)] = z
            return carry
        lax.fori_loop(0, N // 16, zbody, 0)

        fire(0, 0)
        fire(1, 1)

        def pbody(pi, carry):
            for b in range(2):
                cb = 2 * pi + b
                wait(cb, b)

                def ebody(k, carry2, b=b, cb=cb):
                    sl = pl.ds(cb * CW + k * 16, 16)
                    wsl = pl.ds(k * 16, 16)
                    si = src_v[sl]
                    di = dst_v[sl]
                    o = [plsc.load_gather(outs[i], [si]) for i in range(DH)]
                    for j in range(DH):
                        msg = o[0] * webs[b][j, wsl]
                        for i in range(1, DH):
                            msg = msg + o[i] * webs[b][i * DH + j, wsl]
                        plsc.addupdate_scatter(
                            agg_v, [jnp.full((16,), j, jnp.int32), di], msg)
                    return carry2
                lax.fori_loop(0, CW // 16, ebody, 0)

                @pl.when(cb + 2 < NCH)
                def _(cb=cb, b=b):
                    fire(cb + 2, b)
            return carry
        lax.fori_loop(0, NCH // 2, pbody, 0)

        pltpu.sync_copy(agg_v, agg_sh.at[s])
        plsc.subcore_barrier()

        for j in range(DH):
            pltpu.sync_copy(agg_sh.at[:, j, pl.ds(nbase, NPT)], red_v)

            def rbody(k, carry, j=j):
                sl = pl.ds(k * 16, 16)
                acc = red_v[0, sl]
                for t in range(1, NS):
                    acc = acc + red_v[t, sl]
                acc_v[j, sl] = acc
                return carry
            lax.fori_loop(0, NPT // 16, rbody, 0)

        A = [[_splat(prm_v, k2 * DH + j2) for j2 in range(DH)]
             for k2 in range(DH)]
        B = [[_splat(prm_v, 16 + k2 * DH + j2) for j2 in range(DH)]
             for k2 in range(DH)]
        cb = [_splat(prm_v, 32 + j2) for j2 in range(DH)]
        mb = [_splat(prm_v, 36 + j2) for j2 in range(DH)]

        def ubody(k, carry):
            sl = pl.ds(k * 16, 16)
            gsl = pl.ds(nbase + k * 16, 16)
            o = [outs[i][gsl] for i in range(DH)]
            m = [jnp.maximum(acc_v[j2, sl] + o[j2] + cb[j2], 0.0)
                 for j2 in range(DH)]
            for j2 in range(DH):
                nv = mb[j2]
                for k2 in range(DH):
                    nv = nv + m[k2] * A[k2][j2] + o[k2] * B[k2][j2]
                if last:
                    nv = nv + x_v[j2, sl]
                new_v[j2, sl] = nv
            return carry
        lax.fori_loop(0, NPT // 16, ubody, 0)

        if last:
            pltpu.sync_copy(new_v, sf_hbm.at[c, :, pl.ds(nbase, NPT)])
        else:
            pltpu.sync_copy(new_v, out_sh.at[:, pl.ds(nbase, NPT)])
            plsc.subcore_barrier()
            for j in range(DH):
                pltpu.sync_copy(out_sh.at[j], outs[j])


def _message_passing(out0, xT, weT, eidx, prm):
    mesh = plsc.VectorSubcoreMesh(core_axis_name="c", subcore_axis_name="s",
                                  num_cores=2, num_subcores=NS)
    f32 = jnp.float32
    kern = pl.kernel(
        _mp_body,
        out_type=jax.ShapeDtypeStruct((2, DH, N), f32),
        mesh=mesh,
        compiler_params=pltpu.CompilerParams(needs_layout_passes=False),
        scratch_types=[
            pltpu.VMEM((EPT,), jnp.int32),
            pltpu.VMEM((EPT,), jnp.int32),
            pltpu.VMEM((16, CW), f32),
            pltpu.VMEM((16, CW), f32),
            pltpu.VMEM((N,), f32),
            pltpu.VMEM((N,), f32),
            pltpu.VMEM((N,), f32),
            pltpu.VMEM((N,), f32),
            pltpu.VMEM((DH, N), f32),
            pltpu.VMEM((NS, NPT), f32),
            pltpu.VMEM((DH, NPT), f32),
            pltpu.VMEM((DH, NPT), f32),
            pltpu.VMEM((DH, NPT), f32),
            pltpu.VMEM((64,), f32),
            pltpu.SemaphoreType.DMA,
            pltpu.SemaphoreType.DMA,
            pltpu.VMEM_SHARED((NS, DH, N), f32),
            pltpu.VMEM_SHARED((DH, N), f32),
        ],
    )
    return kern(out0, xT, weT, eidx, prm)


def _imap_body(sf_ref, vfT_ref, vf_ref, slmT_ref, vlm_ref,
               sp_ref, vp_ref, vp_acc):
    i = pl.program_id(0)
    j = pl.program_id(1)
    t = jnp.tanh(jnp.dot(sf_ref[...], vfT_ref[...],
                         preferred_element_type=jnp.float32))
    t = t * (slmT_ref[...] * vlm_ref[...])
    sp_blk = jnp.dot(t, vf_ref[...], preferred_element_type=jnp.float32)

    @pl.when(j == 0)
    def _():
        sp_ref[...] = sp_blk

    @pl.when(j != 0)
    def _():
        sp_ref[...] = sp_ref[...] + sp_blk

    vp_blk = lax.dot_general(t, sf_ref[...], (((0,), (0,)), ((), ())),
                             preferred_element_type=jnp.float32)

    @pl.when(i == 0)
    def _():
        vp_acc[pl.ds(j * BN, BN), :] = vp_blk

    @pl.when(i != 0)
    def _():
        vp_acc[pl.ds(j * BN, BN), :] = vp_acc[pl.ds(j * BN, BN), :] + vp_blk

    @pl.when((i == N // BM - 1) & (j == N // BN - 1))
    def _():
        vp_ref[...] = vp_acc[...]


def _interact(sf, vfT, vf, slmT, vlm):
    grid = (N // BM, N // BN)
    return pl.pallas_call(
        _imap_body,
        grid=grid,
        in_specs=[
            pl.BlockSpec((BM, DH), lambda i, j: (i, 0)),
            pl.BlockSpec((DH, BN), lambda i, j: (0, j)),
            pl.BlockSpec((BN, DH), lambda i, j: (j, 0)),
            pl.BlockSpec((BM, 1), lambda i, j: (i, 0)),
            pl.BlockSpec((1, BN), lambda i, j: (0, j)),
        ],
        out_specs=[
            pl.BlockSpec((BM, DH), lambda i, j: (i, 0)),
            pl.BlockSpec((N, DH), lambda i, j: (0, 0)),
        ],
        out_shape=[
            jax.ShapeDtypeStruct((N, DH), jnp.float32),
            jax.ShapeDtypeStruct((N, DH), jnp.float32),
        ],
        scratch_shapes=[pltpu.VMEM((N, DH), jnp.float32)],
    )(sf, vfT, vf, slmT, vlm)


def _dot_t(a, b):
    return lax.dot_general(a, b, (((1,), (1,)), ((), ())),
                           preferred_element_type=jnp.float32)


def _set2set(feat, wih, whh, bih, bhh):
    d = 2 * DH
    q_star = jnp.zeros((1, 2 * d), jnp.float32)
    h = jnp.zeros((1, d), jnp.float32)
    c = jnp.zeros((1, d), jnp.float32)
    for _ in range(2):
        gates = _dot_t(q_star, wih) + bih + _dot_t(h, whh) + bhh
        ig = jax.nn.sigmoid(gates[:, 0:d])
        fg = jax.nn.sigmoid(gates[:, d:2 * d])
        gg = jnp.tanh(gates[:, 2 * d:3 * d])
        og = jax.nn.sigmoid(gates[:, 3 * d:4 * d])
        c = fg * c + ig * gg
        h = og * jnp.tanh(c)
        e = _dot_t(feat, h)
        mx = jnp.max(e, axis=0, keepdims=True)
        al = jnp.exp(e - mx)
        al = al / jnp.sum(al, axis=0, keepdims=True)
        readout = lax.dot_general(al, feat, (((0,), (0,)), ((), ())),
                                  preferred_element_type=jnp.float32)
        q_star = jnp.concatenate([h, readout], axis=1)
    return q_star


def _tail_body(sf_ref, sp_ref, vf_ref, vp_ref, wih_ref, whh_ref,
               bih_ref, bhh_ref, fc1w_ref, fc1b_ref, fc2w_ref, fc2b_ref,
               out_ref):
    sf2 = jnp.concatenate([sf_ref[...], sp_ref[...]], axis=1)
    vf2 = jnp.concatenate([vf_ref[...], vp_ref[...]], axis=1)
    ss = _set2set(sf2, wih_ref[0], whh_ref[0], bih_ref[0], bhh_ref[0])
    sv = _set2set(vf2, wih_ref[1], whh_ref[1], bih_ref[1], bhh_ref[1])
    data = jnp.concatenate([ss, sv], axis=1)
    data = jnp.maximum(_dot_t(data, fc1w_ref[...]) + fc1b_ref[...], 0.0)
    out_ref[...] = (jnp.sum(data * fc2w_ref[...], axis=1, keepdims=True)
                    + fc2b_ref[...])


def _tail(sf, sp, vf, vp, wih, whh, bih, bhh, fc1w, fc1b, fc2w, fc2b):
    return pl.pallas_call(
        _tail_body,
        out_shape=jax.ShapeDtypeStruct((1, 1), jnp.float32),
    )(sf, sp, vf, vp, wih, whh, bih, bhh, fc1w, fc1b, fc2w, fc2b)


def _prm_vec(p, pre):
    w = p[pre + "msg_W"]
    a = w[:, :DH].T.reshape(-1)
    b = w[:, DH:].T.reshape(-1)
    return jnp.concatenate([
        a, b, p[pre + "conv_bias"], p[pre + "msg_b"],
        jnp.zeros((24,), jnp.float32)])


def kernel(solute_x, solute_edge_index, solute_e, solvent_x,
           solvent_edge_index, solvent_e, solute_len_matrix,
           solvent_len_matrix, params):
    p = params
    eT = jnp.stack([solute_e.T, solvent_e.T])
    w1 = jnp.stack([p["su_en1_W"], p["sv_en1_W"]])
    b1 = jnp.stack([p["su_en1_b"], p["sv_en1_b"]])[..., None]
    w2 = jnp.stack([p["su_en2_W"], p["sv_en2_W"]])
    b2 = jnp.stack([p["su_en2_b"], p["sv_en2_b"]])[..., None]
    weT = _edge_net(eT, w1, b1, w2, b2)

    xT = jnp.stack([solute_x.T, solvent_x.T])
    l0w = jnp.stack([p["su_lin0_W"], p["sv_lin0_W"]])
    l0b = jnp.stack([p["su_lin0_b"], p["sv_lin0_b"]])[..., None]
    out0 = _lin0(xT, l0w, l0b)

    eidx = jnp.stack([solute_edge_index, solvent_edge_index])
    prm = jnp.stack([_prm_vec(p, "su_"), _prm_vec(p, "sv_")])
    sfT = _message_passing(out0, xT, weT, eidx, prm)

    sf = sfT[0].T
    vf = sfT[1].T
    sp, vp = _interact(sf, sfT[1], vf, solute_len_matrix.T,
                       solvent_len_matrix)

    wih = jnp.stack([p["s2s_su_W_ih"], p["s2s_sv_W_ih"]])
    whh = jnp.stack([p["s2s_su_W_hh"], p["s2s_sv_W_hh"]])
    bih = jnp.stack([p["s2s_su_b_ih"], p["s2s_sv_b_ih"]])[:, None, :]
    bhh = jnp.stack([p["s2s_su_b_hh"], p["s2s_sv_b_hh"]])[:, None, :]
    return _tail(sf, sp, vf, vp, wih, whh, bih, bhh,
                 p["fc1_W"], p["fc1_b"][None, :], p["fc2_W"],
                 p["fc2_b"][None, :])


# speedup vs baseline: 13.6492x; 1.0602x over previous
import functools

import jax
import jax.numpy as jnp
from jax import lax
from jax.experimental import pallas as pl
from jax.experimental.pallas import tpu as pltpu
from jax.experimental.pallas import tpu_sc as plsc

N = 4096
E = 65536
DH = 4
NSTEP = 6
NS = 16
EPT = E // NS
NPT = N // NS
CW = 1024
NCH = EPT // CW
BE = 2048
BM = 1024
BN = 1024


def _edge_net_body(eT_ref, w1_ref, b1_ref, w2_ref, b2_ref, out_ref):
    h = jnp.maximum(
        jnp.dot(w1_ref[0], eT_ref[0], preferred_element_type=jnp.float32)
        + b1_ref[0], 0.0)
    out_ref[0] = (
        jnp.dot(w2_ref[0], h, preferred_element_type=jnp.float32) + b2_ref[0])


def _edge_net(eT, w1, b1, w2, b2):
    grid = (2, E // BE)
    return pl.pallas_call(
        _edge_net_body,
        grid=grid,
        in_specs=[
            pl.BlockSpec((1, 16, BE), lambda g, j: (g, 0, j)),
            pl.BlockSpec((1, 1024, 16), lambda g, j: (g, 0, 0)),
            pl.BlockSpec((1, 1024, 1), lambda g, j: (g, 0, 0)),
            pl.BlockSpec((1, 16, 1024), lambda g, j: (g, 0, 0)),
            pl.BlockSpec((1, 16, 1), lambda g, j: (g, 0, 0)),
        ],
        out_specs=pl.BlockSpec((1, 16, BE), lambda g, j: (g, 0, j)),
        out_shape=jax.ShapeDtypeStruct((2, 16, E), jnp.float32),
    )(eT, w1, b1, w2, b2)


def _lin0_body(xT_ref, w_ref, b_ref, out_ref):
    out_ref[0] = jnp.maximum(
        jnp.dot(w_ref[0], xT_ref[0], preferred_element_type=jnp.float32)
        + b_ref[0], 0.0)


def _lin0(xT, w, b):
    return pl.pallas_call(
        _lin0_body,
        grid=(2,),
        in_specs=[
            pl.BlockSpec((1, DH, N), lambda g: (g, 0, 0)),
            pl.BlockSpec((1, DH, DH), lambda g: (g, 0, 0)),
            pl.BlockSpec((1, DH, 1), lambda g: (g, 0, 0)),
        ],
        out_specs=pl.BlockSpec((1, DH, N), lambda g: (g, 0, 0)),
        out_shape=jax.ShapeDtypeStruct((2, DH, N), jnp.float32),
    )(xT, w, b)


def _mp_body(out0_hbm, xT_hbm, weT_hbm, eidx_hbm, prm_hbm, sf_hbm,
             src_v, dst_v, web0, web1, o0, o1, o2, o3, agg_v,
             red_v, acc_v, new_v, x_v, prm_v, sem0, sem1, agg_sh, out_sh):
    c = lax.axis_index("c")
    s = lax.axis_index("s")
    ebase = s * EPT
    nbase = s * NPT
    outs = [o0, o1, o2, o3]
    webs = [web0, web1]
    sems = [sem0, sem1]

    pltpu.sync_copy(eidx_hbm.at[c, 0, pl.ds(ebase, EPT)], src_v)
    pltpu.sync_copy(eidx_hbm.at[c, 1, pl.ds(ebase, EPT)], dst_v)
    for j in range(DH):
        pltpu.sync_copy(out0_hbm.at[c, j], outs[j])
    pltpu.sync_copy(xT_hbm.at[c, :, pl.ds(nbase, NPT)], x_v)
    pltpu.sync_copy(prm_hbm.at[c], prm_v)

    def fire(cb, b):
        pltpu.async_copy(
            weT_hbm.at[c, :, pl.ds(ebase + cb * CW, CW)], webs[b], sems[b])

    def wait(cb, b):
        pltpu.make_async_copy(
            weT_hbm.at[c, :, pl.ds(ebase + cb * CW, CW)], webs[b],
            sems[b]).wait()

    for step in range(NSTEP):
        last = step == NSTEP - 1

        def zbody(k, carry):
            z = jnp.zeros((16,), jnp.float32)
            for j in range(DH):
                agg_v[j, pl.ds(k * 16, 16)] = z
            return carry
        lax.fori_loop(0, N // 16, zbody, 0)

        fire(0, 0)
        fire(1, 1)

        def pbody(pi, carry):
            for b in range(2):
                cb = 2 * pi + b
                wait(cb, b)

                def ebody(k, carry2, b=b, cb=cb):
                    sl = pl.ds(cb * CW + k * 16, 16)
                    wsl = pl.ds(k * 16, 16)
                    si = src_v[sl]
                    di = dst_v[sl]
                    o = [plsc.load_gather(outs[i], [si]) for i in range(DH)]
                    for j in range(DH):
                        msg = o[0] * webs[b][j, wsl]
                        for i in range(1, DH):
                            msg = msg + o[i] * webs[b][i * DH + j, wsl]
                        plsc.addupdate_scatter(
                            agg_v, [jnp.full((16,), j, jnp.int32), di], msg)
                    return carry2
                lax.fori_loop(0, CW // 16, ebody, 0)

                @pl.when(cb + 2 < NCH)
                def _(cb=cb, b=b):
                    fire(cb + 2, b)
            return carry
        lax.fori_loop(0, NCH // 2, pbody, 0)

        pltpu.sync_copy(agg_v, agg_sh.at[s])
        plsc.subcore_barrier()

        for j in range(DH):
            pltpu.sync_copy(agg_sh.at[:, j, pl.ds(nbase, NPT)], red_v)

            def rbody(k, carry, j=j):
                sl = pl.ds(k * 16, 16)
                acc = red_v[0, sl]
                for t in range(1, NS):
                    acc = acc + red_v[t, sl]
                acc_v[j, sl] = acc
                return carry
            lax.fori_loop(0, NPT // 16, rbody, 0)

        A = [[prm_v[k2 * DH + j2] for j2 in range(DH)] for k2 in range(DH)]
        B = [[prm_v[16 + k2 * DH + j2] for j2 in range(DH)]
             for k2 in range(DH)]
        cb = [prm_v[32 + j2] for j2 in range(DH)]
        mb = [prm_v[36 + j2] for j2 in range(DH)]

        def ubody(k, carry):
            sl = pl.ds(k * 16, 16)
            gsl = pl.ds(nbase + k * 16, 16)
            o = [outs[i][gsl] for i in range(DH)]
            m = [jnp.maximum(acc_v[j2, sl] + o[j2] + cb[j2], 0.0)
                 for j2 in range(DH)]
            for j2 in range(DH):
                nv = mb[j2]
                for k2 in range(DH):
                    nv = nv + m[k2] * A[k2][j2] + o[k2] * B[k2][j2]
                if last:
                    nv = nv + x_v[j2, sl]
                new_v[j2, sl] = nv
            return carry
        lax.fori_loop(0, NPT // 16, ubody, 0)

        if last:
            pltpu.sync_copy(new_v, sf_hbm.at[c, :, pl.ds(nbase, NPT)])
        else:
            pltpu.sync_copy(new_v, out_sh.at[:, pl.ds(nbase, NPT)])
            plsc.subcore_barrier()
            for j in range(DH):
                pltpu.sync_copy(out_sh.at[j], outs[j])


def _message_passing(out0, xT, weT, eidx, prm):
    mesh = plsc.VectorSubcoreMesh(core_axis_name="c", subcore_axis_name="s",
                                  num_cores=2, num_subcores=NS)
    f32 = jnp.float32
    kern = pl.kernel(
        _mp_body,
        out_type=jax.ShapeDtypeStruct((2, DH, N), f32),
        mesh=mesh,
        compiler_params=pltpu.CompilerParams(needs_layout_passes=False),
        scratch_types=[
            pltpu.VMEM((EPT,), jnp.int32),
            pltpu.VMEM((EPT,), jnp.int32),
            pltpu.VMEM((16, CW), f32),
            pltpu.VMEM((16, CW), f32),
            pltpu.VMEM((N,), f32),
            pltpu.VMEM((N,), f32),
            pltpu.VMEM((N,), f32),
            pltpu.VMEM((N,), f32),
            pltpu.VMEM((DH, N), f32),
            pltpu.VMEM((NS, NPT), f32),
            pltpu.VMEM((DH, NPT), f32),
            pltpu.VMEM((DH, NPT), f32),
            pltpu.VMEM((DH, NPT), f32),
            pltpu.VMEM((40, 16), f32),
            pltpu.SemaphoreType.DMA,
            pltpu.SemaphoreType.DMA,
            pltpu.VMEM_SHARED((NS, DH, N), f32),
            pltpu.VMEM_SHARED((DH, N), f32),
        ],
    )
    return kern(out0, xT, weT, eidx, prm)


def _imap_body(sf_ref, vfT_ref, vf_ref, slmT_ref, vlm_ref,
               sp_ref, vp_ref, vp_acc):
    i = pl.program_id(0)
    j = pl.program_id(1)
    t = jnp.tanh(jnp.dot(sf_ref[...], vfT_ref[...],
                         preferred_element_type=jnp.float32))
    t = t * (slmT_ref[...] * vlm_ref[...])
    sp_blk = jnp.dot(t, vf_ref[...], preferred_element_type=jnp.float32)

    @pl.when(j == 0)
    def _():
        sp_ref[...] = sp_blk

    @pl.when(j != 0)
    def _():
        sp_ref[...] = sp_ref[...] + sp_blk

    vp_blk = lax.dot_general(t, sf_ref[...], (((0,), (0,)), ((), ())),
                             preferred_element_type=jnp.float32)

    @pl.when(i == 0)
    def _():
        vp_acc[pl.ds(j * BN, BN), :] = vp_blk

    @pl.when(i != 0)
    def _():
        vp_acc[pl.ds(j * BN, BN), :] = vp_acc[pl.ds(j * BN, BN), :] + vp_blk

    @pl.when((i == N // BM - 1) & (j == N // BN - 1))
    def _():
        vp_ref[...] = vp_acc[...]


def _interact(sf, vfT, vf, slmT, vlm):
    grid = (N // BM, N // BN)
    return pl.pallas_call(
        _imap_body,
        grid=grid,
        in_specs=[
            pl.BlockSpec((BM, DH), lambda i, j: (i, 0)),
            pl.BlockSpec((DH, BN), lambda i, j: (0, j)),
            pl.BlockSpec((BN, DH), lambda i, j: (j, 0)),
            pl.BlockSpec((BM, 1), lambda i, j: (i, 0)),
            pl.BlockSpec((1, BN), lambda i, j: (0, j)),
        ],
        out_specs=[
            pl.BlockSpec((BM, DH), lambda i, j: (i, 0)),
            pl.BlockSpec((N, DH), lambda i, j: (0, 0)),
        ],
        out_shape=[
            jax.ShapeDtypeStruct((N, DH), jnp.float32),
            jax.ShapeDtypeStruct((N, DH), jnp.float32),
        ],
        scratch_shapes=[pltpu.VMEM((N, DH), jnp.float32)],
    )(sf, vfT, vf, slmT, vlm)


def _dot_t(a, b):
    return lax.dot_general(a, b, (((1,), (1,)), ((), ())),
                           preferred_element_type=jnp.float32)


def _set2set(feat, wih, whh, bih, bhh):
    d = 2 * DH
    q_star = jnp.zeros((1, 2 * d), jnp.float32)
    h = jnp.zeros((1, d), jnp.float32)
    c = jnp.zeros((1, d), jnp.float32)
    for _ in range(2):
        gates = _dot_t(q_star, wih) + bih + _dot_t(h, whh) + bhh
        ig = jax.nn.sigmoid(gates[:, 0:d])
        fg = jax.nn.sigmoid(gates[:, d:2 * d])
        gg = jnp.tanh(gates[:, 2 * d:3 * d])
        og = jax.nn.sigmoid(gates[:, 3 * d:4 * d])
        c = fg * c + ig * gg
        h = og * jnp.tanh(c)
        e = _dot_t(feat, h)
        mx = jnp.max(e, axis=0, keepdims=True)
        al = jnp.exp(e - mx)
        al = al / jnp.sum(al, axis=0, keepdims=True)
        readout = lax.dot_general(al, feat, (((0,), (0,)), ((), ())),
                                  preferred_element_type=jnp.float32)
        q_star = jnp.concatenate([h, readout], axis=1)
    return q_star


def _tail_body(sf_ref, sp_ref, vf_ref, vp_ref, wih_ref, whh_ref,
               bih_ref, bhh_ref, fc1w_ref, fc1b_ref, fc2w_ref, fc2b_ref,
               out_ref):
    sf2 = jnp.concatenate([sf_ref[...], sp_ref[...]], axis=1)
    vf2 = jnp.concatenate([vf_ref[...], vp_ref[...]], axis=1)
    ss = _set2set(sf2, wih_ref[0], whh_ref[0], bih_ref[0], bhh_ref[0])
    sv = _set2set(vf2, wih_ref[1], whh_ref[1], bih_ref[1], bhh_ref[1])
    data = jnp.concatenate([ss, sv], axis=1)
    data = jnp.maximum(_dot_t(data, fc1w_ref[...]) + fc1b_ref[...], 0.0)
    out_ref[...] = (jnp.sum(data * fc2w_ref[...], axis=1, keepdims=True)
                    + fc2b_ref[...])


def _tail(sf, sp, vf, vp, wih, whh, bih, bhh, fc1w, fc1b, fc2w, fc2b):
    return pl.pallas_call(
        _tail_body,
        out_shape=jax.ShapeDtypeStruct((1, 1), jnp.float32),
    )(sf, sp, vf, vp, wih, whh, bih, bhh, fc1w, fc1b, fc2w, fc2b)


def _prm_vec(p, pre):
    w = p[pre + "msg_W"]
    a = w[:, :DH].T.reshape(-1)
    b = w[:, DH:].T.reshape(-1)
    v = jnp.concatenate([a, b, p[pre + "conv_bias"], p[pre + "msg_b"]])
    return jnp.broadcast_to(v[:, None], (40, 16))


def kernel(solute_x, solute_edge_index, solute_e, solvent_x,
           solvent_edge_index, solvent_e, solute_len_matrix,
           solvent_len_matrix, params):
    p = params
    eT = jnp.stack([solute_e.T, solvent_e.T])
    w1 = jnp.stack([p["su_en1_W"], p["sv_en1_W"]])
    b1 = jnp.stack([p["su_en1_b"], p["sv_en1_b"]])[..., None]
    w2 = jnp.stack([p["su_en2_W"], p["sv_en2_W"]])
    b2 = jnp.stack([p["su_en2_b"], p["sv_en2_b"]])[..., None]
    weT = _edge_net(eT, w1, b1, w2, b2)

    xT = jnp.stack([solute_x.T, solvent_x.T])
    l0w = jnp.stack([p["su_lin0_W"], p["sv_lin0_W"]])
    l0b = jnp.stack([p["su_lin0_b"], p["sv_lin0_b"]])[..., None]
    out0 = _lin0(xT, l0w, l0b)

    eidx = jnp.stack([solute_edge_index, solvent_edge_index])
    prm = jnp.stack([_prm_vec(p, "su_"), _prm_vec(p, "sv_")])
    sfT = _message_passing(out0, xT, weT, eidx, prm)

    sf = sfT[0].T
    vf = sfT[1].T
    sp, vp = _interact(sf, sfT[1], vf, solute_len_matrix.T,
                       solvent_len_matrix)

    wih = jnp.stack([p["s2s_su_W_ih"], p["s2s_sv_W_ih"]])
    whh = jnp.stack([p["s2s_su_W_hh"], p["s2s_sv_W_hh"]])
    bih = jnp.stack([p["s2s_su_b_ih"], p["s2s_sv_b_ih"]])[:, None, :]
    bhh = jnp.stack([p["s2s_su_b_hh"], p["s2s_sv_b_hh"]])[:, None, :]
    return _tail(sf, sp, vf, vp, wih, whh, bih, bhh,
                 p["fc1_W"], p["fc1_b"][None, :], p["fc2_W"],
                 p["fc2_b"][None, :])


# speedup vs baseline: 13.6824x; 1.0024x over previous
import jax
import jax.numpy as jnp
from jax import lax
from jax.experimental import pallas as pl
from jax.experimental.pallas import tpu as pltpu
from jax.experimental.pallas import tpu_sc as plsc

N = 4096
E = 65536
DH = 4
NSTEP = 6
NS = 16
EPT = E // NS
NPT = N // NS
CW = 1024
NCH = EPT // CW
BE = 2048
BM = 1024
BN = 1024


def _edge_net_body(eT_ref, w1_ref, b1_ref, w2_ref, b2_ref, out_ref):
    h = jnp.maximum(
        jnp.dot(w1_ref[0], eT_ref[0], preferred_element_type=jnp.float32)
        + b1_ref[0], 0.0)
    out_ref[0] = (
        jnp.dot(w2_ref[0], h, preferred_element_type=jnp.float32) + b2_ref[0])


def _edge_net(eT, w1, b1, w2, b2):
    grid = (2, E // BE)
    return pl.pallas_call(
        _edge_net_body,
        grid=grid,
        in_specs=[
            pl.BlockSpec((1, 16, BE), lambda g, j: (g, 0, j)),
            pl.BlockSpec((1, 1024, 16), lambda g, j: (g, 0, 0)),
            pl.BlockSpec((1, 1024, 1), lambda g, j: (g, 0, 0)),
            pl.BlockSpec((1, 16, 1024), lambda g, j: (g, 0, 0)),
            pl.BlockSpec((1, 16, 1), lambda g, j: (g, 0, 0)),
        ],
        out_specs=pl.BlockSpec((1, 16, BE), lambda g, j: (g, 0, j)),
        out_shape=jax.ShapeDtypeStruct((2, 16, E), jnp.float32),
    )(eT, w1, b1, w2, b2)


def _lin0_body(xT_ref, w_ref, b_ref, out_ref):
    out_ref[0] = jnp.maximum(
        jnp.dot(w_ref[0], xT_ref[0], preferred_element_type=jnp.float32)
        + b_ref[0], 0.0)


def _lin0(xT, w, b):
    return pl.pallas_call(
        _lin0_body,
        grid=(2,),
        in_specs=[
            pl.BlockSpec((1, DH, N), lambda g: (g, 0, 0)),
            pl.BlockSpec((1, DH, DH), lambda g: (g, 0, 0)),
            pl.BlockSpec((1, DH, 1), lambda g: (g, 0, 0)),
        ],
        out_specs=pl.BlockSpec((1, DH, N), lambda g: (g, 0, 0)),
        out_shape=jax.ShapeDtypeStruct((2, DH, N), jnp.float32),
    )(xT, w, b)


def _mp_body(out0_hbm, xT_hbm, weT_hbm, eidx_hbm, prm_hbm, sf_hbm,
             src_v, dst_v, web0, web1, o0, o1, o2, o3, agg_v,
             red_v, acc_v, new_v, x_v, prm_v, sem0, sem1, agg_sh, out_sh):
    c = lax.axis_index("c")
    s = lax.axis_index("s")
    ebase = s * EPT
    nbase = s * NPT
    outs = [o0, o1, o2, o3]
    webs = [web0, web1]
    sems = [sem0, sem1]

    pltpu.sync_copy(eidx_hbm.at[c, 0, pl.ds(ebase, EPT)], src_v)
    pltpu.sync_copy(eidx_hbm.at[c, 1, pl.ds(ebase, EPT)], dst_v)
    for j in range(DH):
        pltpu.sync_copy(out0_hbm.at[c, j], outs[j])
    pltpu.sync_copy(xT_hbm.at[c, :, pl.ds(nbase, NPT)], x_v)
    pltpu.sync_copy(prm_hbm.at[c], prm_v)

    def fire(cb, b):
        pltpu.async_copy(
            weT_hbm.at[c, :, pl.ds(ebase + cb * CW, CW)], webs[b], sems[b])

    def wait(cb, b):
        pltpu.make_async_copy(
            weT_hbm.at[c, :, pl.ds(ebase + cb * CW, CW)], webs[b],
            sems[b]).wait()

    for step in range(NSTEP):
        last = step == NSTEP - 1

        def zbody(k, carry):
            z = jnp.zeros((16,), jnp.float32)
            for j in range(DH):
                agg_v[j, pl.ds(k * 16, 16)] = z
            return carry
        lax.fori_loop(0, N // 16, zbody, 0)

        fire(0, 0)
        fire(1, 1)

        def pbody(pi, carry):
            for b in range(2):
                cb = 2 * pi + b
                wait(cb, b)

                def ebody(k, carry2, b=b, cb=cb):
                    sl = pl.ds(cb * CW + k * 16, 16)
                    wsl = pl.ds(k * 16, 16)
                    si = src_v[sl]
                    di = dst_v[sl]
                    o = [plsc.load_gather(outs[i], [si]) for i in range(DH)]
                    for j in range(DH):
                        msg = o[0] * webs[b][j, wsl]
                        for i in range(1, DH):
                            msg = msg + o[i] * webs[b][i * DH + j, wsl]
                        plsc.addupdate_scatter(
                            agg_v, [jnp.full((16,), j, jnp.int32), di], msg)
                    return carry2
                lax.fori_loop(0, CW // 16, ebody, 0)

                @pl.when(cb + 2 < NCH)
                def _(cb=cb, b=b):
                    fire(cb + 2, b)
            return carry
        lax.fori_loop(0, NCH // 2, pbody, 0)

        pltpu.sync_copy(agg_v, agg_sh.at[s])
        plsc.subcore_barrier()

        for j in range(DH):
            pltpu.sync_copy(agg_sh.at[:, j, pl.ds(nbase, NPT)], red_v)

            def rbody(k, carry, j=j):
                sl = pl.ds(k * 16, 16)
                acc = red_v[0, sl]
                for t in range(1, NS):
                    acc = acc + red_v[t, sl]
                acc_v[j, sl] = acc
                return carry
            lax.fori_loop(0, NPT // 16, rbody, 0)

        A = [[prm_v[k2 * DH + j2] for j2 in range(DH)] for k2 in range(DH)]
        B = [[prm_v[16 + k2 * DH + j2] for j2 in range(DH)]
             for k2 in range(DH)]
        cb = [prm_v[32 + j2] for j2 in range(DH)]
        mb = [prm_v[36 + j2] for j2 in range(DH)]

        def ubody(k, carry):
            sl = pl.ds(k * 16, 16)
            gsl = pl.ds(nbase + k * 16, 16)
            o = [outs[i][gsl] for i in range(DH)]
            m = [jnp.maximum(acc_v[j2, sl] + o[j2] + cb[j2], 0.0)
                 for j2 in range(DH)]
            for j2 in range(DH):
                nv = mb[j2]
                for k2 in range(DH):
                    nv = nv + m[k2] * A[k2][j2] + o[k2] * B[k2][j2]
                if last:
                    nv = nv + x_v[j2, sl]
                new_v[j2, sl] = nv
            return carry
        lax.fori_loop(0, NPT // 16, ubody, 0)

        if last:
            pltpu.sync_copy(new_v, sf_hbm.at[c, :, pl.ds(nbase, NPT)])
        else:
            pltpu.sync_copy(new_v, out_sh.at[:, pl.ds(nbase, NPT)])
            plsc.subcore_barrier()
            for j in range(DH):
                pltpu.sync_copy(out_sh.at[j], outs[j])


def _message_passing(out0, xT, weT, eidx, prm):
    mesh = plsc.VectorSubcoreMesh(core_axis_name="c", subcore_axis_name="s",
                                  num_cores=2, num_subcores=NS)
    f32 = jnp.float32
    kern = pl.kernel(
        _mp_body,
        out_type=jax.ShapeDtypeStruct((2, DH, N), f32),
        mesh=mesh,
        compiler_params=pltpu.CompilerParams(needs_layout_passes=False),
        scratch_types=[
            pltpu.VMEM((EPT,), jnp.int32),
            pltpu.VMEM((EPT,), jnp.int32),
            pltpu.VMEM((16, CW), f32),
            pltpu.VMEM((16, CW), f32),
            pltpu.VMEM((N,), f32),
            pltpu.VMEM((N,), f32),
            pltpu.VMEM((N,), f32),
            pltpu.VMEM((N,), f32),
            pltpu.VMEM((DH, N), f32),
            pltpu.VMEM((NS, NPT), f32),
            pltpu.VMEM((DH, NPT), f32),
            pltpu.VMEM((DH, NPT), f32),
            pltpu.VMEM((DH, NPT), f32),
            pltpu.VMEM((40, 16), f32),
            pltpu.SemaphoreType.DMA,
            pltpu.SemaphoreType.DMA,
            pltpu.VMEM_SHARED((NS, DH, N), f32),
            pltpu.VMEM_SHARED((DH, N), f32),
        ],
    )
    return kern(out0, xT, weT, eidx, prm)


def _imap_body(sf_ref, vfT_ref, vf_ref, slmT_ref, vlm_ref,
               sp_ref, vp_ref, vp_acc):
    i = pl.program_id(0)
    j = pl.program_id(1)
    t = jnp.tanh(jnp.dot(sf_ref[...], vfT_ref[...],
                         preferred_element_type=jnp.float32))
    t = t * (slmT_ref[...] * vlm_ref[...])
    sp_blk = jnp.dot(t, vf_ref[...], preferred_element_type=jnp.float32)

    @pl.when(j == 0)
    def _():
        sp_ref[...] = sp_blk

    @pl.when(j != 0)
    def _():
        sp_ref[...] = sp_ref[...] + sp_blk

    vp_blk = lax.dot_general(t, sf_ref[...], (((0,), (0,)), ((), ())),
                             preferred_element_type=jnp.float32)

    @pl.when(i == 0)
    def _():
        vp_acc[pl.ds(j * BN, BN), :] = vp_blk

    @pl.when(i != 0)
    def _():
        vp_acc[pl.ds(j * BN, BN), :] = vp_acc[pl.ds(j * BN, BN), :] + vp_blk

    @pl.when((i == N // BM - 1) & (j == N // BN - 1))
    def _():
        vp_ref[...] = vp_acc[...]


def _interact(sf, vfT, vf, slmT, vlm):
    grid = (N // BM, N // BN)
    return pl.pallas_call(
        _imap_body,
        grid=grid,
        in_specs=[
            pl.BlockSpec((BM, DH), lambda i, j: (i, 0)),
            pl.BlockSpec((DH, BN), lambda i, j: (0, j)),
            pl.BlockSpec((BN, DH), lambda i, j: (j, 0)),
            pl.BlockSpec((BM, 1), lambda i, j: (i, 0)),
            pl.BlockSpec((1, BN), lambda i, j: (0, j)),
        ],
        out_specs=[
            pl.BlockSpec((BM, DH), lambda i, j: (i, 0)),
            pl.BlockSpec((N, DH), lambda i, j: (0, 0)),
        ],
        out_shape=[
            jax.ShapeDtypeStruct((N, DH), jnp.float32),
            jax.ShapeDtypeStruct((N, DH), jnp.float32),
        ],
        scratch_shapes=[pltpu.VMEM((N, DH), jnp.float32)],
    )(sf, vfT, vf, slmT, vlm)


def _dot_t(a, b):
    return lax.dot_general(a, b, (((1,), (1,)), ((), ())),
                           preferred_element_type=jnp.float32)


def _set2set(feat, wih, whh, bih, bhh):
    d = 2 * DH
    q_star = jnp.zeros((1, 2 * d), jnp.float32)
    h = jnp.zeros((1, d), jnp.float32)
    c = jnp.zeros((1, d), jnp.float32)
    for _ in range(2):
        gates = _dot_t(q_star, wih) + bih + _dot_t(h, whh) + bhh
        ig = jax.nn.sigmoid(gates[:, 0:d])
        fg = jax.nn.sigmoid(gates[:, d:2 * d])
        gg = jnp.tanh(gates[:, 2 * d:3 * d])
        og = jax.nn.sigmoid(gates[:, 3 * d:4 * d])
        c = fg * c + ig * gg
        h = og * jnp.tanh(c)
        e = _dot_t(feat, h)
        mx = jnp.max(e, axis=0, keepdims=True)
        al = jnp.exp(e - mx)
        al = al / jnp.sum(al, axis=0, keepdims=True)
        readout = lax.dot_general(al, feat, (((0,), (0,)), ((), ())),
                                  preferred_element_type=jnp.float32)
        q_star = jnp.concatenate([h, readout], axis=1)
    return q_star


def _tail_body(sf_ref, sp_ref, vf_ref, vp_ref, wih_ref, whh_ref,
               bih_ref, bhh_ref, fc1w_ref, fc1b_ref, fc2w_ref, fc2b_ref,
               out_ref):
    sf2 = jnp.concatenate([sf_ref[...], sp_ref[...]], axis=1)
    vf2 = jnp.concatenate([vf_ref[...], vp_ref[...]], axis=1)
    ss = _set2set(sf2, wih_ref[0], whh_ref[0], bih_ref[0], bhh_ref[0])
    sv = _set2set(vf2, wih_ref[1], whh_ref[1], bih_ref[1], bhh_ref[1])
    data = jnp.concatenate([ss, sv], axis=1)
    data = jnp.maximum(_dot_t(data, fc1w_ref[...]) + fc1b_ref[...], 0.0)
    out_ref[...] = (jnp.sum(data * fc2w_ref[...], axis=1, keepdims=True)
                    + fc2b_ref[...])


def _tail(sf, sp, vf, vp, wih, whh, bih, bhh, fc1w, fc1b, fc2w, fc2b):
    return pl.pallas_call(
        _tail_body,
        out_shape=jax.ShapeDtypeStruct((1, 1), jnp.float32),
    )(sf, sp, vf, vp, wih, whh, bih, bhh, fc1w, fc1b, fc2w, fc2b)


def _prm_vec(p, pre):
    w = p[pre + "msg_W"]
    a = w[:, :DH].T.reshape(-1)
    b = w[:, DH:].T.reshape(-1)
    v = jnp.concatenate([a, b, p[pre + "conv_bias"], p[pre + "msg_b"]])
    return jnp.broadcast_to(v[:, None], (40, 16))


def kernel(solute_x, solute_edge_index, solute_e, solvent_x,
           solvent_edge_index, solvent_e, solute_len_matrix,
           solvent_len_matrix, params):
    p = params
    eT = jnp.stack([solute_e.T, solvent_e.T])
    w1 = jnp.stack([p["su_en1_W"], p["sv_en1_W"]])
    b1 = jnp.stack([p["su_en1_b"], p["sv_en1_b"]])[..., None]
    w2 = jnp.stack([p["su_en2_W"], p["sv_en2_W"]])
    b2 = jnp.stack([p["su_en2_b"], p["sv_en2_b"]])[..., None]
    weT = _edge_net(eT, w1, b1, w2, b2)

    xT = jnp.stack([solute_x.T, solvent_x.T])
    l0w = jnp.stack([p["su_lin0_W"], p["sv_lin0_W"]])
    l0b = jnp.stack([p["su_lin0_b"], p["sv_lin0_b"]])[..., None]
    out0 = _lin0(xT, l0w, l0b)

    eidx = jnp.stack([solute_edge_index, solvent_edge_index])
    prm = jnp.stack([_prm_vec(p, "su_"), _prm_vec(p, "sv_")])
    sfT = _message_passing(out0, xT, weT, eidx, prm)

    sf = sfT[0].T
    vf = sfT[1].T
    sp, vp = _interact(sf, sfT[1], vf, solute_len_matrix.T,
                       solvent_len_matrix)

    wih = jnp.stack([p["s2s_su_W_ih"], p["s2s_sv_W_ih"]])
    whh = jnp.stack([p["s2s_su_W_hh"], p["s2s_sv_W_hh"]])
    bih = jnp.stack([p["s2s_su_b_ih"], p["s2s_sv_b_ih"]])[:, None, :]
    bhh = jnp.stack([p["s2s_su_b_hh"], p["s2s_sv_b_hh"]])[:, None, :]
    return _tail(sf, sp, vf, vp, wih, whh, bih, bhh,
                 p["fc1_W"], p["fc1_b"][None, :], p["fc2_W"],
                 p["fc2_b"][None, :])
